# baseline (speedup 1.0000x reference)
import functools

import jax
import jax.numpy as jnp
import numpy as np
from jax import lax
from jax.experimental import pallas as pl
from jax.experimental.pallas import tpu as pltpu
from jax.experimental.pallas import tpu_sc as plsc

B, L = 16384, 50
VOCAB_, EMB = 100000, 64
FC, OUT_ = 256, 64

NW = 32
BAGS_PER_W = B // NW
CHUNK_BAGS = 2
IDX_PER_CHUNK = CHUNK_BAGS * L
N_CHUNKS = BAGS_PER_W // CHUNK_BAGS
NGRP = EMB // 16


NBUF = 8


def _sc_body(idx_hbm, table_hbm, out_hbm, idx_v, g0, g1, g2, g3, g4, g5,
             g6, g7, outbuf, s0, s1, s2, s3, s4, s5, s6, s7):
    gbufs = (g0, g1, g2, g3, g4, g5, g6, g7)
    sems = (s0, s1, s2, s3, s4, s5, s6, s7)
    w = lax.axis_index("s") * 2 + lax.axis_index("c")
    pltpu.sync_copy(idx_hbm.at[w], idx_v)

    def accumulate(gbuf, j):
        for b in range(CHUNK_BAGS):
            accs = [None] * NGRP
            for r in range(L):
                for c in range(EMB // 32):
                    x = gbuf[b * L + r, pl.ds(c * 32, 32)]
                    lo, hi = plsc.unpack(
                        x, format=plsc.PackFormat.INTERLEAVED,
                        preferred_element_type=jnp.float32)
                    if r == 0:
                        accs[2 * c], accs[2 * c + 1] = lo, hi
                    else:
                        accs[2 * c] = accs[2 * c] + lo
                        accs[2 * c + 1] = accs[2 * c + 1] + hi
            for g in range(NGRP):
                outbuf[j * CHUNK_BAGS + b, pl.ds(g * 16, 16)] = accs[g]

    for p in range(NBUF):
        pltpu.async_copy(table_hbm.at[idx_v.at[p]], gbufs[p], sems[p])

    def body(jj, carry):
        j0 = NBUF * jj
        for p in range(NBUF):
            j = j0 + p
            pltpu.make_async_copy(
                table_hbm.at[idx_v.at[j]], gbufs[p], sems[p]).wait()
            accumulate(gbufs[p], j)

            @pl.when(j + NBUF < N_CHUNKS)
            def _():
                pltpu.async_copy(
                    table_hbm.at[idx_v.at[j + NBUF]], gbufs[p], sems[p])

        return carry

    lax.fori_loop(0, N_CHUNKS // NBUF, body, 0)
    pltpu.sync_copy(outbuf, out_hbm.at[pl.ds(w * BAGS_PER_W, BAGS_PER_W)])


_sc_bag_sums = functools.partial(
    pl.kernel,
    mesh=plsc.VectorSubcoreMesh(core_axis_name="c", subcore_axis_name="s"),
    out_type=jax.ShapeDtypeStruct((B, EMB), jnp.float32),
    scratch_types=[
        pltpu.VMEM((N_CHUNKS, IDX_PER_CHUNK), jnp.int32),
        *([pltpu.VMEM((IDX_PER_CHUNK, EMB), jnp.bfloat16)] * 8),
        pltpu.VMEM((BAGS_PER_W, EMB), jnp.float32),
        *([pltpu.SemaphoreType.DMA] * 8),
    ],
    compiler_params=pltpu.CompilerParams(
        use_tc_tiling_on_sc=False, needs_layout_passes=False),
)(_sc_body)


def _mlp_body(sums_ref, text_ref, w1_ref, b1_ref, w2_ref, b2_ref, out_ref):
    mask = (text_ref[...] != 0).astype(jnp.float32)
    cnt = jnp.maximum(jnp.sum(mask, axis=1, keepdims=True), 1.0)
    bag = sums_ref[...] / cnt
    h = lax.dot_general(bag, w1_ref[...], (((1,), (1,)), ((), ())),
                        preferred_element_type=jnp.float32)
    h = jnp.maximum(h + b1_ref[...], 0.0)
    o = lax.dot_general(h, w2_ref[...], (((1,), (1,)), ((), ())),
                        preferred_element_type=jnp.float32)
    out_ref[...] = o + b2_ref[...]


TB = 1024


def _mlp(sums, text, W1, b1, W2, b2):
    return pl.pallas_call(
        _mlp_body,
        grid=(B // TB,),
        in_specs=[
            pl.BlockSpec((TB, EMB), lambda i: (i, 0)),
            pl.BlockSpec((TB, L), lambda i: (i, 0)),
            pl.BlockSpec((FC, EMB), lambda i: (0, 0)),
            pl.BlockSpec((1, FC), lambda i: (0, 0)),
            pl.BlockSpec((OUT_, FC), lambda i: (0, 0)),
            pl.BlockSpec((1, OUT_), lambda i: (0, 0)),
        ],
        out_specs=pl.BlockSpec((TB, OUT_), lambda i: (i, 0)),
        out_shape=jax.ShapeDtypeStruct((B, OUT_), jnp.float32),
    )(sums, text, W1, b1.reshape(1, FC), W2, b2.reshape(1, OUT_))


_PERM = np.concatenate([
    np.concatenate([32 * c + 2 * np.arange(16) + p for p in (0, 1)])
    for c in range(EMB // 32)
])


def kernel(text, emb_table, W1, b1, W2, b2):
    text = text.astype(jnp.int32)
    idx = text.reshape(NW, N_CHUNKS, IDX_PER_CHUNK)
    sums = _sc_bag_sums(idx, emb_table.astype(jnp.bfloat16))
    return _mlp(sums, text, W1[:, _PERM], b1, W2, b2)


# speedup vs baseline: 1.2806x; 1.2806x over previous
import functools

import jax
import jax.numpy as jnp
import numpy as np
from jax import lax
from jax.experimental import pallas as pl
from jax.experimental.pallas import tpu as pltpu
from jax.experimental.pallas import tpu_sc as plsc

B, L = 16384, 50
VOCAB_, EMB = 100000, 64
FC, OUT_ = 256, 64

NW = 32
BAGS_PER_W = B // NW
CHUNK_BAGS = 1
IDX_PER_CHUNK = CHUNK_BAGS * L
N_CHUNKS = BAGS_PER_W // CHUNK_BAGS
NGRP = EMB // 16


NBUF = 8


def _sc_body(idx_hbm, table_hbm, out_hbm, idx_v, g0, g1, g2, g3, g4, g5,
             g6, g7, outbuf, s0, s1, s2, s3, s4, s5, s6, s7):
    gbufs = (g0, g1, g2, g3, g4, g5, g6, g7)
    sems = (s0, s1, s2, s3, s4, s5, s6, s7)
    w = lax.axis_index("s") * 2 + lax.axis_index("c")
    pltpu.sync_copy(idx_hbm.at[w], idx_v)

    def accumulate(gbuf, j):
        for b in range(CHUNK_BAGS):
            accs = [None] * NGRP
            for r in range(L):
                for c in range(EMB // 32):
                    x = gbuf[b * L + r, pl.ds(c * 32, 32)]
                    lo, hi = plsc.unpack(
                        x, format=plsc.PackFormat.INTERLEAVED,
                        preferred_element_type=jnp.float32)
                    if r == 0:
                        accs[2 * c], accs[2 * c + 1] = lo, hi
                    else:
                        accs[2 * c] = accs[2 * c] + lo
                        accs[2 * c + 1] = accs[2 * c + 1] + hi
            for g in range(NGRP):
                outbuf[j * CHUNK_BAGS + b, pl.ds(g * 16, 16)] = accs[g]

    for p in range(NBUF):
        pltpu.async_copy(table_hbm.at[idx_v.at[p]], gbufs[p], sems[p])

    def body(jj, carry):
        j0 = NBUF * jj
        for p in range(NBUF):
            j = j0 + p
            pltpu.make_async_copy(
                table_hbm.at[idx_v.at[j]], gbufs[p], sems[p]).wait()
            accumulate(gbufs[p], j)

            @pl.when(j + NBUF < N_CHUNKS)
            def _():
                pltpu.async_copy(
                    table_hbm.at[idx_v.at[j + NBUF]], gbufs[p], sems[p])

        return carry

    lax.fori_loop(0, N_CHUNKS // NBUF, body, 0)
    pltpu.sync_copy(outbuf, out_hbm.at[pl.ds(w * BAGS_PER_W, BAGS_PER_W)])


_sc_bag_sums = functools.partial(
    pl.kernel,
    mesh=plsc.VectorSubcoreMesh(core_axis_name="c", subcore_axis_name="s"),
    out_type=jax.ShapeDtypeStruct((B, EMB), jnp.float32),
    scratch_types=[
        pltpu.VMEM((N_CHUNKS, IDX_PER_CHUNK), jnp.int32),
        *([pltpu.VMEM((IDX_PER_CHUNK, EMB), jnp.bfloat16)] * 8),
        pltpu.VMEM((BAGS_PER_W, EMB), jnp.float32),
        *([pltpu.SemaphoreType.DMA] * 8),
    ],
    compiler_params=pltpu.CompilerParams(
        use_tc_tiling_on_sc=False, needs_layout_passes=False),
)(_sc_body)


def _mlp_body(sums_ref, text_ref, w1_ref, b1_ref, w2_ref, b2_ref, out_ref):
    mask = (text_ref[...] != 0).astype(jnp.float32)
    cnt = jnp.maximum(jnp.sum(mask, axis=1, keepdims=True), 1.0)
    bag = sums_ref[...] / cnt
    h = lax.dot_general(bag, w1_ref[...], (((1,), (1,)), ((), ())),
                        preferred_element_type=jnp.float32)
    h = jnp.maximum(h + b1_ref[...], 0.0)
    o = lax.dot_general(h, w2_ref[...], (((1,), (1,)), ((), ())),
                        preferred_element_type=jnp.float32)
    out_ref[...] = o + b2_ref[...]


TB = 1024


def _mlp(sums, text, W1, b1, W2, b2):
    return pl.pallas_call(
        _mlp_body,
        grid=(B // TB,),
        in_specs=[
            pl.BlockSpec((TB, EMB), lambda i: (i, 0)),
            pl.BlockSpec((TB, L), lambda i: (i, 0)),
            pl.BlockSpec((FC, EMB), lambda i: (0, 0)),
            pl.BlockSpec((1, FC), lambda i: (0, 0)),
            pl.BlockSpec((OUT_, FC), lambda i: (0, 0)),
            pl.BlockSpec((1, OUT_), lambda i: (0, 0)),
        ],
        out_specs=pl.BlockSpec((TB, OUT_), lambda i: (i, 0)),
        out_shape=jax.ShapeDtypeStruct((B, OUT_), jnp.float32),
    )(sums, text, W1, b1.reshape(1, FC), W2, b2.reshape(1, OUT_))


_PERM = np.concatenate([
    np.concatenate([32 * c + 2 * np.arange(16) + p for p in (0, 1)])
    for c in range(EMB // 32)
])


def kernel(text, emb_table, W1, b1, W2, b2):
    text = text.astype(jnp.int32)
    idx = text.reshape(NW, N_CHUNKS, IDX_PER_CHUNK)
    sums = _sc_bag_sums(idx, emb_table.astype(jnp.bfloat16))
    return _mlp(sums, text, W1[:, _PERM], b1, W2, b2)
